# baseline (speedup 1.0000x reference)
import functools

import jax
import jax.numpy as jnp
from jax import lax
from jax.experimental import pallas as pl
from jax.experimental.pallas import tpu as pltpu
from jax.experimental.pallas import tpu_sc as plsc

_NC = 2
_NS = 16
_NW = _NC * _NS
_L = 16

_C = 80
_D = 128
_KD = _D // _L


def _rsqrt_nr(s):
    i = lax.bitcast_convert_type(s, jnp.int32)
    i = jnp.int32(0x5F3759DF) - lax.shift_right_arithmetic(i, 1)
    y = lax.bitcast_convert_type(i, jnp.float32)
    for _ in range(2):
        h = (jnp.float32(0.5) * s) * y
        y = y * (jnp.float32(1.5) - h * y)
    return y


def _make_sc_kernel(n_edges):
    chunks_pw = n_edges // (_NW * _C)
    pairs = (chunks_pw - 1) // 2
    mesh = plsc.VectorSubcoreMesh(core_axis_name="c", subcore_axis_name="s")

    @functools.partial(
        pl.kernel,
        mesh=mesh,
        compiler_params=pltpu.CompilerParams(needs_layout_passes=False),
        out_type=(
            jax.ShapeDtypeStruct((n_edges,), jnp.float32),
            jax.ShapeDtypeStruct((n_edges, _D), jnp.float32),
        ),
        scratch_types=[
            pltpu.VMEM((1, chunks_pw, _C), jnp.int32),
            pltpu.VMEM((1, chunks_pw, _C), jnp.int32),
            pltpu.VMEM((_C, _D), jnp.float32),
            pltpu.VMEM((_C, _D), jnp.float32),
            pltpu.VMEM((_C, _D), jnp.float32),
            pltpu.VMEM((_C, _D), jnp.float32),
            pltpu.VMEM((_C,), jnp.float32),
            pltpu.VMEM((_C,), jnp.float32),
            pltpu.VMEM((_L * _L,), jnp.float32),
            pltpu.SemaphoreType.DMA,
            pltpu.SemaphoreType.DMA,
            pltpu.SemaphoreType.DMA,
        ],
    )
    def k(x_hbm, srci_hbm, dsti_hbm, dist_hbm, dir_hbm,
          idx_s, idx_d, src_a, src_b, dst_a, dst_b, dist_a, dist_b, rcpb,
          sem_gs, sem_gd, sem_w):
        wid = lax.axis_index("s") * _NC + lax.axis_index("c")
        base_edge = wid * chunks_pw * _C

        pltpu.sync_copy(srci_hbm.at[pl.ds(wid, 1)], idx_s)
        pltpu.sync_copy(dsti_hbm.at[pl.ds(wid, 1)], idx_d)

        lane = lax.iota(jnp.int32, _L)

        def issue_gather(c, sb, db):
            pltpu.async_copy(x_hbm.at[idx_s.at[0, c]], sb, sem_gs)
            pltpu.async_copy(x_hbm.at[idx_d.at[0, c]], db, sem_gd)

        def wait_gather(c, sb, db):
            pltpu.make_async_copy(x_hbm.at[idx_s.at[0, c]], sb, sem_gs).wait()
            pltpu.make_async_copy(x_hbm.at[idx_d.at[0, c]], db, sem_gd).wait()

        def issue_writes(c, sb, tb):
            off = base_edge + c * _C
            pltpu.async_copy(sb, dir_hbm.at[pl.ds(off, _C)], sem_w)
            pltpu.async_copy(tb, dist_hbm.at[pl.ds(off, _C)], sem_w)

        def wait_writes(c, sb, tb):
            off = base_edge + c * _C
            pltpu.make_async_copy(sb, dir_hbm.at[pl.ds(off, _C)], sem_w).wait()
            pltpu.make_async_copy(
                tb, dist_hbm.at[pl.ds(off, _C)], sem_w).wait()

        def compute(c, sb, db, tb):
            for g in range(_C // _L):
                g16 = g * _L

                def e_a(j, s_g, g16=g16):
                    e = g16 + j
                    acc = jnp.zeros((_L,), jnp.float32)
                    for kk in range(_KD):
                        sv = sb[e, pl.ds(kk * _L, _L)]
                        dv = db[e, pl.ds(kk * _L, _L)]
                        r = sv - dv
                        sb[e, pl.ds(kk * _L, _L)] = r
                        acc = acc + r * r
                    totv = jnp.full((_L,), jnp.sum(acc))
                    return jnp.where(lane == j, totv, s_g)

                s_g = lax.fori_loop(
                    0, _L, e_a, jnp.zeros((_L,), jnp.float32), unroll=4)

                dist_g = s_g * _rsqrt_nr(s_g)
                tb[pl.ds(g16, _L)] = dist_g
                rcp_g = jnp.float32(1.0) / (jnp.float32(1.0) + dist_g)
                for j in range(_L):
                    rcpb[pl.ds(j * _L, _L)] = jnp.full((_L,), rcp_g[j])

                def e_b(j, carry, g16=g16):
                    e = g16 + j
                    rcv = rcpb[pl.ds(j * _L, _L)]
                    for kk in range(_KD):
                        sb[e, pl.ds(kk * _L, _L)] = (
                            sb[e, pl.ds(kk * _L, _L)] * rcv)
                    return carry

                lax.fori_loop(0, _L, e_b, 0, unroll=2)
            issue_writes(c, sb, tb)

        issue_gather(0, src_a, dst_a)

        def pair(j, carry):
            c0 = 2 * j
            wait_gather(c0, src_a, dst_a)

            @pl.when(j > 0)
            def _():
                wait_writes(c0 - 1, src_b, dist_b)

            issue_gather(c0 + 1, src_b, dst_b)
            compute(c0, src_a, dst_a, dist_a)
            wait_gather(c0 + 1, src_b, dst_b)
            wait_writes(c0, src_a, dist_a)
            issue_gather(c0 + 2, src_a, dst_a)
            compute(c0 + 1, src_b, dst_b, dist_b)
            return carry

        lax.fori_loop(0, pairs, pair, 0)

        last = chunks_pw - 1
        wait_gather(last, src_a, dst_a)
        wait_writes(last - 1, src_b, dist_b)
        compute(last, src_a, dst_a, dist_a)
        wait_writes(last, src_a, dist_a)

    return k


def kernel(x, edge_index):
    n_edges = edge_index.shape[1]
    srci = edge_index[0].astype(jnp.int32).reshape(_NW, -1, _C)
    dsti = edge_index[1].astype(jnp.int32).reshape(_NW, -1, _C)
    dist, edir = _make_sc_kernel(n_edges)(x, srci, dsti)
    return dist, edir


# speedup vs baseline: 1.2671x; 1.2671x over previous
import functools

import jax
import jax.numpy as jnp
from jax import lax
from jax.experimental import pallas as pl
from jax.experimental.pallas import tpu as pltpu
from jax.experimental.pallas import tpu_sc as plsc

_NC = 2
_NS = 16
_NW = _NC * _NS
_L = 16

_C = 80
_D = 128
_KD = _D // _L


def _rsqrt_nr(s):
    i = lax.bitcast_convert_type(s, jnp.int32)
    i = jnp.int32(0x5F3759DF) - lax.shift_right_arithmetic(i, 1)
    y = lax.bitcast_convert_type(i, jnp.float32)
    for _ in range(2):
        h = (jnp.float32(0.5) * s) * y
        y = y * (jnp.float32(1.5) - h * y)
    return y


def _make_sc_kernel(n_edges):
    chunks_pw = n_edges // (_NW * _C)
    pairs = (chunks_pw - 1) // 2
    mesh = plsc.VectorSubcoreMesh(core_axis_name="c", subcore_axis_name="s")

    @functools.partial(
        pl.kernel,
        mesh=mesh,
        compiler_params=pltpu.CompilerParams(needs_layout_passes=False),
        out_type=(
            jax.ShapeDtypeStruct((n_edges,), jnp.float32),
            jax.ShapeDtypeStruct((n_edges, _D), jnp.float32),
        ),
        scratch_types=[
            pltpu.VMEM((1, chunks_pw, _C), jnp.int32),
            pltpu.VMEM((1, chunks_pw, _C), jnp.int32),
            pltpu.VMEM((_C, _D), jnp.float32),
            pltpu.VMEM((_C, _D), jnp.float32),
            pltpu.VMEM((_C, _D), jnp.float32),
            pltpu.VMEM((_C, _D), jnp.float32),
            pltpu.VMEM((_C,), jnp.float32),
            pltpu.VMEM((_C,), jnp.float32),
            pltpu.VMEM((_L * _L,), jnp.float32),
            pltpu.SemaphoreType.DMA,
            pltpu.SemaphoreType.DMA,
            pltpu.SemaphoreType.DMA,
        ],
    )
    def k(x_hbm, srci_hbm, dsti_hbm, dist_hbm, dir_hbm,
          idx_s, idx_d, src_a, src_b, dst_a, dst_b, dist_a, dist_b, rcpb,
          sem_gs, sem_gd, sem_w):
        wid = lax.axis_index("s") * _NC + lax.axis_index("c")
        base_edge = wid * chunks_pw * _C

        pltpu.sync_copy(srci_hbm.at[pl.ds(wid, 1)], idx_s)
        pltpu.sync_copy(dsti_hbm.at[pl.ds(wid, 1)], idx_d)

        lane = lax.iota(jnp.int32, _L)

        def issue_gather(c, sb, db):
            pltpu.async_copy(x_hbm.at[idx_s.at[0, c]], sb, sem_gs)
            pltpu.async_copy(x_hbm.at[idx_d.at[0, c]], db, sem_gd)

        def wait_gather(c, sb, db):
            pltpu.make_async_copy(x_hbm.at[idx_s.at[0, c]], sb, sem_gs).wait()
            pltpu.make_async_copy(x_hbm.at[idx_d.at[0, c]], db, sem_gd).wait()

        def issue_writes(c, sb, tb):
            off = base_edge + c * _C
            pltpu.async_copy(sb, dir_hbm.at[pl.ds(off, _C)], sem_w)
            pltpu.async_copy(tb, dist_hbm.at[pl.ds(off, _C)], sem_w)

        def wait_writes(c, sb, tb):
            off = base_edge + c * _C
            pltpu.make_async_copy(sb, dir_hbm.at[pl.ds(off, _C)], sem_w).wait()
            pltpu.make_async_copy(
                tb, dist_hbm.at[pl.ds(off, _C)], sem_w).wait()

        def compute(c, sb, db, tb):
            for g in range(_C // _L):
                g16 = g * _L

                def e_a(j, s_g, g16=g16):
                    e = g16 + j
                    acc = jnp.zeros((_L,), jnp.float32)
                    for kk in range(_KD):
                        sv = sb[e, pl.ds(kk * _L, _L)]
                        dv = db[e, pl.ds(kk * _L, _L)]
                        r = sv - dv
                        sb[e, pl.ds(kk * _L, _L)] = r
                        acc = acc + r * r
                    totv = jnp.full((_L,), jnp.sum(acc))
                    return jnp.where(lane == j, totv, s_g)

                s_g = lax.fori_loop(
                    0, _L, e_a, jnp.zeros((_L,), jnp.float32))

                dist_g = s_g * _rsqrt_nr(s_g)
                tb[pl.ds(g16, _L)] = dist_g
                rcp_g = jnp.float32(1.0) / (jnp.float32(1.0) + dist_g)
                for j in range(_L):
                    rcpb[pl.ds(j * _L, _L)] = jnp.full((_L,), rcp_g[j])

                def e_b(j, carry, g16=g16):
                    e = g16 + j
                    rcv = rcpb[pl.ds(j * _L, _L)]
                    for kk in range(_KD):
                        sb[e, pl.ds(kk * _L, _L)] = (
                            sb[e, pl.ds(kk * _L, _L)] * rcv)
                    return carry

                lax.fori_loop(0, _L, e_b, 0)
            issue_writes(c, sb, tb)

        issue_gather(0, src_a, dst_a)

        def pair(j, carry):
            c0 = 2 * j
            wait_gather(c0, src_a, dst_a)

            @pl.when(j > 0)
            def _():
                wait_writes(c0 - 1, src_b, dist_b)

            issue_gather(c0 + 1, src_b, dst_b)
            compute(c0, src_a, dst_a, dist_a)
            wait_gather(c0 + 1, src_b, dst_b)
            wait_writes(c0, src_a, dist_a)
            issue_gather(c0 + 2, src_a, dst_a)
            compute(c0 + 1, src_b, dst_b, dist_b)
            return carry

        lax.fori_loop(0, pairs, pair, 0)

        last = chunks_pw - 1
        wait_gather(last, src_a, dst_a)
        wait_writes(last - 1, src_b, dist_b)
        compute(last, src_a, dst_a, dist_a)
        wait_writes(last, src_a, dist_a)

    return k


def kernel(x, edge_index):
    n_edges = edge_index.shape[1]
    srci = edge_index[0].astype(jnp.int32).reshape(_NW, -1, _C)
    dsti = edge_index[1].astype(jnp.int32).reshape(_NW, -1, _C)
    dist, edir = _make_sc_kernel(n_edges)(x, srci, dsti)
    return dist, edir


# speedup vs baseline: 1.3777x; 1.0873x over previous
import functools

import jax
import jax.numpy as jnp
from jax import lax
from jax.experimental import pallas as pl
from jax.experimental.pallas import tpu as pltpu
from jax.experimental.pallas import tpu_sc as plsc

_NC = 2
_NS = 16
_NW = _NC * _NS
_L = 16

_C = 80
_D = 128
_KD = _D // _L


def _rsqrt_nr(s):
    i = lax.bitcast_convert_type(s, jnp.int32)
    i = jnp.int32(0x5F3759DF) - lax.shift_right_arithmetic(i, 1)
    y = lax.bitcast_convert_type(i, jnp.float32)
    for _ in range(2):
        h = (jnp.float32(0.5) * s) * y
        y = y * (jnp.float32(1.5) - h * y)
    return y


def _make_sc_kernel(n_edges):
    chunks_pw = n_edges // (_NW * _C)
    pairs = (chunks_pw - 1) // 2
    mesh = plsc.VectorSubcoreMesh(core_axis_name="c", subcore_axis_name="s")

    @functools.partial(
        pl.kernel,
        mesh=mesh,
        compiler_params=pltpu.CompilerParams(needs_layout_passes=False),
        out_type=(
            jax.ShapeDtypeStruct((n_edges,), jnp.float32),
            jax.ShapeDtypeStruct((n_edges, _D), jnp.float32),
        ),
        scratch_types=[
            pltpu.VMEM((1, chunks_pw, _C), jnp.int32),
            pltpu.VMEM((1, chunks_pw, _C), jnp.int32),
            pltpu.VMEM((_C, _D), jnp.float32),
            pltpu.VMEM((_C, _D), jnp.float32),
            pltpu.VMEM((_C, _D), jnp.float32),
            pltpu.VMEM((_C, _D), jnp.float32),
            pltpu.VMEM((_C,), jnp.float32),
            pltpu.VMEM((_C,), jnp.float32),
            pltpu.VMEM((_L * _L,), jnp.float32),
            pltpu.SemaphoreType.DMA,
            pltpu.SemaphoreType.DMA,
            pltpu.SemaphoreType.DMA,
        ],
    )
    def k(x_hbm, srci_hbm, dsti_hbm, dist_hbm, dir_hbm,
          idx_s, idx_d, src_a, src_b, dst_a, dst_b, dist_a, dist_b, rcpb,
          sem_gs, sem_gd, sem_w):
        wid = lax.axis_index("s") * _NC + lax.axis_index("c")
        base_edge = wid * chunks_pw * _C

        pltpu.sync_copy(srci_hbm.at[pl.ds(wid, 1)], idx_s)
        pltpu.sync_copy(dsti_hbm.at[pl.ds(wid, 1)], idx_d)

        lane = lax.iota(jnp.int32, _L)

        def issue_gather(c, sb, db):
            pltpu.async_copy(x_hbm.at[idx_s.at[0, c]], sb, sem_gs)
            pltpu.async_copy(x_hbm.at[idx_d.at[0, c]], db, sem_gd)

        def wait_gather(c, sb, db):
            pltpu.make_async_copy(x_hbm.at[idx_s.at[0, c]], sb, sem_gs).wait()
            pltpu.make_async_copy(x_hbm.at[idx_d.at[0, c]], db, sem_gd).wait()

        def issue_writes(c, sb, tb):
            off = base_edge + c * _C
            pltpu.async_copy(sb, dir_hbm.at[pl.ds(off, _C)], sem_w)
            pltpu.async_copy(tb, dist_hbm.at[pl.ds(off, _C)], sem_w)

        def wait_writes(c, sb, tb):
            off = base_edge + c * _C
            pltpu.make_async_copy(sb, dir_hbm.at[pl.ds(off, _C)], sem_w).wait()
            pltpu.make_async_copy(
                tb, dist_hbm.at[pl.ds(off, _C)], sem_w).wait()

        def compute(c, sb, db, tb):
            for g in range(_C // _L):
                def e_body(j, dist_g, g16=g * _L):
                    e = g16 + j
                    acc = jnp.zeros((_L,), jnp.float32)
                    rs = []
                    for kk in range(_KD):
                        sv = sb[e, pl.ds(kk * _L, _L)]
                        dv = db[e, pl.ds(kk * _L, _L)]
                        r = sv - dv
                        rs.append(r)
                        acc = acc + r * r
                    totv = jnp.full((_L,), jnp.sum(acc))
                    distv = totv * _rsqrt_nr(totv)
                    rcv = jnp.float32(1.0) / (jnp.float32(1.0) + distv)
                    for kk in range(_KD):
                        sb[e, pl.ds(kk * _L, _L)] = rs[kk] * rcv
                    return jnp.where(lane == j, distv, dist_g)

                dist_g = lax.fori_loop(
                    0, _L, e_body, jnp.zeros((_L,), jnp.float32))
                tb[pl.ds(g * _L, _L)] = dist_g
            issue_writes(c, sb, tb)

        issue_gather(0, src_a, dst_a)

        def pair(j, carry):
            c0 = 2 * j
            wait_gather(c0, src_a, dst_a)

            @pl.when(j > 0)
            def _():
                wait_writes(c0 - 1, src_b, dist_b)

            issue_gather(c0 + 1, src_b, dst_b)
            compute(c0, src_a, dst_a, dist_a)
            wait_gather(c0 + 1, src_b, dst_b)
            wait_writes(c0, src_a, dist_a)
            issue_gather(c0 + 2, src_a, dst_a)
            compute(c0 + 1, src_b, dst_b, dist_b)
            return carry

        lax.fori_loop(0, pairs, pair, 0)

        last = chunks_pw - 1
        wait_gather(last, src_a, dst_a)
        wait_writes(last - 1, src_b, dist_b)
        compute(last, src_a, dst_a, dist_a)
        wait_writes(last, src_a, dist_a)

    return k


def kernel(x, edge_index):
    n_edges = edge_index.shape[1]
    srci = edge_index[0].astype(jnp.int32).reshape(_NW, -1, _C)
    dsti = edge_index[1].astype(jnp.int32).reshape(_NW, -1, _C)
    dist, edir = _make_sc_kernel(n_edges)(x, srci, dsti)
    return dist, edir


# speedup vs baseline: 1.3802x; 1.0018x over previous
import functools

import jax
import jax.numpy as jnp
from jax import lax
from jax.experimental import pallas as pl
from jax.experimental.pallas import tpu as pltpu
from jax.experimental.pallas import tpu_sc as plsc

_NC = 2
_NS = 16
_NW = _NC * _NS
_L = 16

_C = 80
_D = 128
_KD = _D // _L


def _rsqrt_nr(s):
    i = lax.bitcast_convert_type(s, jnp.int32)
    i = jnp.int32(0x5F3759DF) - lax.shift_right_arithmetic(i, 1)
    y = lax.bitcast_convert_type(i, jnp.float32)
    for _ in range(2):
        h = (jnp.float32(0.5) * s) * y
        y = y * (jnp.float32(1.5) - h * y)
    return y


def _make_sc_kernel(n_edges):
    chunks_pw = n_edges // (_NW * _C)
    mesh = plsc.VectorSubcoreMesh(core_axis_name="c", subcore_axis_name="s")

    @functools.partial(
        pl.kernel,
        mesh=mesh,
        compiler_params=pltpu.CompilerParams(needs_layout_passes=False),
        out_type=(
            jax.ShapeDtypeStruct((n_edges,), jnp.float32),
            jax.ShapeDtypeStruct((n_edges, _D), jnp.float32),
        ),
        scratch_types=[
            pltpu.VMEM((1, chunks_pw, _C), jnp.int32),
            pltpu.VMEM((1, chunks_pw, _C), jnp.int32),
            pltpu.VMEM((2, _C, _D), jnp.float32),
            pltpu.VMEM((2, _C, _D), jnp.float32),
            pltpu.VMEM((2, _C), jnp.float32),
            pltpu.SemaphoreType.DMA,
            pltpu.SemaphoreType.DMA,
            pltpu.SemaphoreType.DMA,
        ],
    )
    def k(x_hbm, srci_hbm, dsti_hbm, dist_hbm, dir_hbm,
          idx_s, idx_d, srcb, dstb, distb, sem_gs, sem_gd, sem_w):
        wid = lax.axis_index("s") * _NC + lax.axis_index("c")
        base_edge = wid * chunks_pw * _C

        pltpu.sync_copy(srci_hbm.at[pl.ds(wid, 1)], idx_s)
        pltpu.sync_copy(dsti_hbm.at[pl.ds(wid, 1)], idx_d)

        lane = lax.iota(jnp.int32, _L)

        def issue_gather(c, p):
            pltpu.async_copy(x_hbm.at[idx_s.at[0, c]], srcb.at[p], sem_gs)
            pltpu.async_copy(x_hbm.at[idx_d.at[0, c]], dstb.at[p], sem_gd)

        def wait_gather(c, p):
            pltpu.make_async_copy(
                x_hbm.at[idx_s.at[0, c]], srcb.at[p], sem_gs).wait()
            pltpu.make_async_copy(
                x_hbm.at[idx_d.at[0, c]], dstb.at[p], sem_gd).wait()

        def issue_writes(c, p):
            off = base_edge + c * _C
            pltpu.async_copy(srcb.at[p], dir_hbm.at[pl.ds(off, _C)], sem_w)
            pltpu.async_copy(distb.at[p], dist_hbm.at[pl.ds(off, _C)], sem_w)

        def wait_writes(c, p):
            off = base_edge + c * _C
            pltpu.make_async_copy(
                srcb.at[p], dir_hbm.at[pl.ds(off, _C)], sem_w).wait()
            pltpu.make_async_copy(
                distb.at[p], dist_hbm.at[pl.ds(off, _C)], sem_w).wait()

        def compute(p):
            for g in range(_C // _L):
                def e_body(j, dist_g, g16=g * _L):
                    e = g16 + j
                    acc = jnp.zeros((_L,), jnp.float32)
                    rs = []
                    for kk in range(_KD):
                        sv = srcb[p, e, pl.ds(kk * _L, _L)]
                        dv = dstb[p, e, pl.ds(kk * _L, _L)]
                        r = sv - dv
                        rs.append(r)
                        acc = acc + r * r
                    totv = jnp.full((_L,), jnp.sum(acc))
                    distv = totv * _rsqrt_nr(totv)
                    rcv = jnp.float32(1.0) / (jnp.float32(1.0) + distv)
                    for kk in range(_KD):
                        srcb[p, e, pl.ds(kk * _L, _L)] = rs[kk] * rcv
                    return jnp.where(lane == j, distv, dist_g)

                dist_g = lax.fori_loop(
                    0, _L, e_body, jnp.zeros((_L,), jnp.float32))
                distb[p, pl.ds(g * _L, _L)] = dist_g

        issue_gather(0, 0)

        def body(c, carry):
            p = lax.rem(c, 2)
            q = 1 - p
            wait_gather(c, p)

            @pl.when(jnp.logical_and(c > 0, c + 1 < chunks_pw))
            def _():
                wait_writes(c - 1, q)

            @pl.when(c + 1 < chunks_pw)
            def _():
                issue_gather(c + 1, q)

            compute(p)
            issue_writes(c, p)
            return carry

        lax.fori_loop(0, chunks_pw, body, 0)

        last = chunks_pw - 1
        wait_writes(last - 1, lax.rem(last - 1, 2))
        wait_writes(last, lax.rem(last, 2))

    return k


def kernel(x, edge_index):
    n_edges = edge_index.shape[1]
    srci = edge_index[0].astype(jnp.int32).reshape(_NW, -1, _C)
    dsti = edge_index[1].astype(jnp.int32).reshape(_NW, -1, _C)
    dist, edir = _make_sc_kernel(n_edges)(x, srci, dsti)
    return dist, edir


# speedup vs baseline: 1.3879x; 1.0056x over previous
import functools

import jax
import jax.numpy as jnp
from jax import lax
from jax.experimental import pallas as pl
from jax.experimental.pallas import tpu as pltpu
from jax.experimental.pallas import tpu_sc as plsc

_NC = 2
_NS = 16
_NW = _NC * _NS
_L = 16

_C = 80
_D = 128
_KD = _D // _L


def _rsqrt_nr(s):
    i = lax.bitcast_convert_type(s, jnp.int32)
    i = jnp.int32(0x5F3759DF) - lax.shift_right_arithmetic(i, 1)
    y = lax.bitcast_convert_type(i, jnp.float32)
    for _ in range(2):
        h = (jnp.float32(0.5) * s) * y
        y = y * (jnp.float32(1.5) - h * y)
    return y


def _make_sc_kernel(n_edges):
    chunks_pw = n_edges // (_NW * _C)
    mesh = plsc.VectorSubcoreMesh(core_axis_name="c", subcore_axis_name="s")

    @functools.partial(
        pl.kernel,
        mesh=mesh,
        compiler_params=pltpu.CompilerParams(needs_layout_passes=False),
        out_type=(
            jax.ShapeDtypeStruct((n_edges,), jnp.float32),
            jax.ShapeDtypeStruct((n_edges, _D), jnp.float32),
        ),
        scratch_types=[
            pltpu.VMEM((1, chunks_pw, _C), jnp.int32),
            pltpu.VMEM((1, chunks_pw, _C), jnp.int32),
            pltpu.VMEM((2, _C, _D), jnp.float32),
            pltpu.VMEM((2, _C, _D), jnp.float32),
            pltpu.VMEM((2, _C), jnp.float32),
            pltpu.SemaphoreType.DMA,
            pltpu.SemaphoreType.DMA,
            pltpu.SemaphoreType.DMA,
        ],
    )
    def k(x_hbm, srci_hbm, dsti_hbm, dist_hbm, dir_hbm,
          idx_s, idx_d, srcb, dstb, distb, sem_gs, sem_gd, sem_w):
        wid = lax.axis_index("s") * _NC + lax.axis_index("c")
        base_edge = wid * chunks_pw * _C

        pltpu.sync_copy(srci_hbm.at[pl.ds(wid, 1)], idx_s)
        pltpu.sync_copy(dsti_hbm.at[pl.ds(wid, 1)], idx_d)

        lane = lax.iota(jnp.int32, _L)

        def issue_gather(c, p):
            pltpu.async_copy(x_hbm.at[idx_s.at[0, c]], srcb.at[p], sem_gs)
            pltpu.async_copy(x_hbm.at[idx_d.at[0, c]], dstb.at[p], sem_gd)

        def wait_gather(c, p):
            pltpu.make_async_copy(
                x_hbm.at[idx_s.at[0, c]], srcb.at[p], sem_gs).wait()
            pltpu.make_async_copy(
                x_hbm.at[idx_d.at[0, c]], dstb.at[p], sem_gd).wait()

        def issue_writes(c, p):
            off = base_edge + c * _C
            pltpu.async_copy(srcb.at[p], dir_hbm.at[pl.ds(off, _C)], sem_w)
            pltpu.async_copy(distb.at[p], dist_hbm.at[pl.ds(off, _C)], sem_w)

        def wait_writes(c, p):
            off = base_edge + c * _C
            pltpu.make_async_copy(
                srcb.at[p], dir_hbm.at[pl.ds(off, _C)], sem_w).wait()
            pltpu.make_async_copy(
                distb.at[p], dist_hbm.at[pl.ds(off, _C)], sem_w).wait()

        def compute(p):
            for g in range(_C // _L):
                def e_body(j, dist_g, g16=g * _L):
                    e = g16 + j
                    acc = jnp.zeros((_L,), jnp.float32)
                    rs = []
                    for kk in range(_KD):
                        sv = srcb[p, e, pl.ds(kk * _L, _L)]
                        dv = dstb[p, e, pl.ds(kk * _L, _L)]
                        r = sv - dv
                        rs.append(r)
                        acc = acc + r * r
                    totv = jnp.full((_L,), jnp.sum(acc))
                    distv = totv * _rsqrt_nr(totv)
                    rcv = jnp.float32(1.0) / (jnp.float32(1.0) + distv)
                    for kk in range(_KD):
                        srcb[p, e, pl.ds(kk * _L, _L)] = rs[kk] * rcv
                    return jnp.where(lane == j, distv, dist_g)

                dist_g = lax.fori_loop(
                    0, _L, e_body, jnp.zeros((_L,), jnp.float32), unroll=2)
                distb[p, pl.ds(g * _L, _L)] = dist_g

        issue_gather(0, 0)

        def body(c, carry):
            p = lax.rem(c, 2)
            q = 1 - p
            wait_gather(c, p)

            @pl.when(jnp.logical_and(c > 0, c + 1 < chunks_pw))
            def _():
                wait_writes(c - 1, q)

            @pl.when(c + 1 < chunks_pw)
            def _():
                issue_gather(c + 1, q)

            compute(p)
            issue_writes(c, p)
            return carry

        lax.fori_loop(0, chunks_pw, body, 0)

        last = chunks_pw - 1
        wait_writes(last - 1, lax.rem(last - 1, 2))
        wait_writes(last, lax.rem(last, 2))

    return k


def kernel(x, edge_index):
    n_edges = edge_index.shape[1]
    srci = edge_index[0].astype(jnp.int32).reshape(_NW, -1, _C)
    dsti = edge_index[1].astype(jnp.int32).reshape(_NW, -1, _C)
    dist, edir = _make_sc_kernel(n_edges)(x, srci, dsti)
    return dist, edir
